# Initial kernel scaffold; baseline (speedup 1.0000x reference)
import jax
import jax.numpy as jnp
from jax.experimental import pallas as pl


def kernel(x, W, b):
    raise NotImplementedError("write your pallas kernel here")


# speedup vs baseline: 1.6209x; 1.6209x over previous
import functools

import jax
import jax.numpy as jnp
from jax.experimental import pallas as pl

D_MODEL = 768
NUM_EXPERTS = 64
TOP_K = 2

_BLOCK = 1024


def _router_block(x_ref, w_ref, b_ref, logits_ref, idx_ref, gates_ref):
    x = x_ref[...]
    logits = jnp.dot(x, w_ref[...], preferred_element_type=jnp.float32)
    logits = logits + b_ref[...]
    logits_ref[...] = logits

    iota = jax.lax.broadcasted_iota(jnp.int32, logits.shape, 1)
    m1 = jnp.max(logits, axis=1, keepdims=True)
    i1 = jnp.min(jnp.where(logits == m1, iota, NUM_EXPERTS), axis=1,
                 keepdims=True)
    masked = jnp.where(iota == i1, -jnp.inf, logits)
    m2 = jnp.max(masked, axis=1, keepdims=True)
    i2 = jnp.min(jnp.where(masked == m2, iota, NUM_EXPERTS), axis=1,
                 keepdims=True)

    idx_ref[...] = jnp.concatenate([i1, i2], axis=1)
    e2 = jnp.exp(m2 - m1)
    g1 = 1.0 / (1.0 + e2)
    gates_ref[...] = jnp.concatenate([g1, e2 * g1], axis=1)


@jax.jit
def kernel(x, W, b):
    B, S, D = x.shape
    n = B * S
    x2 = x.reshape(n, D)
    b2 = b.reshape(1, NUM_EXPERTS)

    grid = (n // _BLOCK,)
    logits, idx, gates = pl.pallas_call(
        _router_block,
        grid=grid,
        in_specs=[
            pl.BlockSpec((_BLOCK, D), lambda i: (i, 0)),
            pl.BlockSpec((D, NUM_EXPERTS), lambda i: (0, 0)),
            pl.BlockSpec((1, NUM_EXPERTS), lambda i: (0, 0)),
        ],
        out_specs=[
            pl.BlockSpec((_BLOCK, NUM_EXPERTS), lambda i: (i, 0)),
            pl.BlockSpec((_BLOCK, TOP_K), lambda i: (i, 0)),
            pl.BlockSpec((_BLOCK, TOP_K), lambda i: (i, 0)),
        ],
        out_shape=[
            jax.ShapeDtypeStruct((n, NUM_EXPERTS), jnp.float32),
            jax.ShapeDtypeStruct((n, TOP_K), jnp.int32),
            jax.ShapeDtypeStruct((n, TOP_K), jnp.float32),
        ],
    )(x2, W, b2)

    return (logits.reshape(B, S, NUM_EXPERTS),
            idx.reshape(B, S, TOP_K),
            gates.reshape(B, S, TOP_K))


# speedup vs baseline: 1.7784x; 1.0972x over previous
import functools

import jax
import jax.numpy as jnp
from jax.experimental import pallas as pl

D_MODEL = 768
NUM_EXPERTS = 64
TOP_K = 2

_BLOCK = 2048


def _router_block(x_ref, w_ref, b_ref, logits_ref, idx_ref, gates_ref):
    x = x_ref[...]
    logits = jnp.dot(x, w_ref[...], preferred_element_type=jnp.float32)
    logits = logits + b_ref[...]
    logits_ref[...] = logits

    iota = jax.lax.broadcasted_iota(jnp.int32, logits.shape, 1)
    m1 = jnp.max(logits, axis=1, keepdims=True)
    i1 = jnp.min(jnp.where(logits == m1, iota, NUM_EXPERTS), axis=1,
                 keepdims=True)
    masked = jnp.where(iota == i1, -jnp.inf, logits)
    m2 = jnp.max(masked, axis=1, keepdims=True)
    i2 = jnp.min(jnp.where(masked == m2, iota, NUM_EXPERTS), axis=1,
                 keepdims=True)

    idx_ref[...] = jnp.concatenate([i1, i2], axis=1)
    e2 = jnp.exp(m2 - m1)
    g1 = 1.0 / (1.0 + e2)
    gates_ref[...] = jnp.concatenate([g1, e2 * g1], axis=1)


@jax.jit
def kernel(x, W, b):
    B, S, D = x.shape
    n = B * S
    x2 = x.reshape(n, D)
    b2 = b.reshape(1, NUM_EXPERTS)

    grid = (n // _BLOCK,)
    logits, idx, gates = pl.pallas_call(
        _router_block,
        grid=grid,
        in_specs=[
            pl.BlockSpec((_BLOCK, D), lambda i: (i, 0)),
            pl.BlockSpec((D, NUM_EXPERTS), lambda i: (0, 0)),
            pl.BlockSpec((1, NUM_EXPERTS), lambda i: (0, 0)),
        ],
        out_specs=[
            pl.BlockSpec((_BLOCK, NUM_EXPERTS), lambda i: (i, 0)),
            pl.BlockSpec((_BLOCK, TOP_K), lambda i: (i, 0)),
            pl.BlockSpec((_BLOCK, TOP_K), lambda i: (i, 0)),
        ],
        out_shape=[
            jax.ShapeDtypeStruct((n, NUM_EXPERTS), jnp.float32),
            jax.ShapeDtypeStruct((n, TOP_K), jnp.int32),
            jax.ShapeDtypeStruct((n, TOP_K), jnp.float32),
        ],
    )(x2, W, b2)

    return (logits.reshape(B, S, NUM_EXPERTS),
            idx.reshape(B, S, TOP_K),
            gates.reshape(B, S, TOP_K))


# speedup vs baseline: 1.8260x; 1.0268x over previous
import functools

import jax
import jax.numpy as jnp
from jax.experimental import pallas as pl

D_MODEL = 768
NUM_EXPERTS = 64
TOP_K = 2

_BLOCK = 4096


def _router_block(x_ref, w_ref, b_ref, logits_ref, idx_ref, gates_ref):
    x = x_ref[...]
    logits = jnp.dot(x, w_ref[...], preferred_element_type=jnp.float32)
    logits = logits + b_ref[...]
    logits_ref[...] = logits

    iota = jax.lax.broadcasted_iota(jnp.int32, logits.shape, 1)
    m1 = jnp.max(logits, axis=1, keepdims=True)
    i1 = jnp.min(jnp.where(logits == m1, iota, NUM_EXPERTS), axis=1,
                 keepdims=True)
    masked = jnp.where(iota == i1, -jnp.inf, logits)
    m2 = jnp.max(masked, axis=1, keepdims=True)
    i2 = jnp.min(jnp.where(masked == m2, iota, NUM_EXPERTS), axis=1,
                 keepdims=True)

    idx_ref[...] = jnp.concatenate([i1, i2], axis=1)
    e2 = jnp.exp(m2 - m1)
    g1 = 1.0 / (1.0 + e2)
    gates_ref[...] = jnp.concatenate([g1, e2 * g1], axis=1)


@jax.jit
def kernel(x, W, b):
    B, S, D = x.shape
    n = B * S
    x2 = x.reshape(n, D)
    b2 = b.reshape(1, NUM_EXPERTS)

    grid = (n // _BLOCK,)
    logits, idx, gates = pl.pallas_call(
        _router_block,
        grid=grid,
        in_specs=[
            pl.BlockSpec((_BLOCK, D), lambda i: (i, 0)),
            pl.BlockSpec((D, NUM_EXPERTS), lambda i: (0, 0)),
            pl.BlockSpec((1, NUM_EXPERTS), lambda i: (0, 0)),
        ],
        out_specs=[
            pl.BlockSpec((_BLOCK, NUM_EXPERTS), lambda i: (i, 0)),
            pl.BlockSpec((_BLOCK, TOP_K), lambda i: (i, 0)),
            pl.BlockSpec((_BLOCK, TOP_K), lambda i: (i, 0)),
        ],
        out_shape=[
            jax.ShapeDtypeStruct((n, NUM_EXPERTS), jnp.float32),
            jax.ShapeDtypeStruct((n, TOP_K), jnp.int32),
            jax.ShapeDtypeStruct((n, TOP_K), jnp.float32),
        ],
    )(x2, W, b2)

    return (logits.reshape(B, S, NUM_EXPERTS),
            idx.reshape(B, S, TOP_K),
            gates.reshape(B, S, TOP_K))


# speedup vs baseline: 1.8270x; 1.0005x over previous
import functools

import jax
import jax.numpy as jnp
from jax.experimental import pallas as pl
from jax.experimental.pallas import tpu as pltpu

D_MODEL = 768
NUM_EXPERTS = 64
TOP_K = 2

_BLOCK = 4096


def _router_block(x_ref, w_ref, b_ref, logits_ref, idx_ref, gates_ref):
    x = x_ref[...]
    logits = jnp.dot(x, w_ref[...], preferred_element_type=jnp.float32)
    logits = logits + b_ref[...]
    logits_ref[...] = logits

    iota = jax.lax.broadcasted_iota(jnp.int32, logits.shape, 1)
    m1 = jnp.max(logits, axis=1, keepdims=True)
    i1 = jnp.min(jnp.where(logits == m1, iota, NUM_EXPERTS), axis=1,
                 keepdims=True)
    masked = jnp.where(iota == i1, -jnp.inf, logits)
    m2 = jnp.max(masked, axis=1, keepdims=True)
    i2 = jnp.min(jnp.where(masked == m2, iota, NUM_EXPERTS), axis=1,
                 keepdims=True)

    idx_ref[...] = jnp.concatenate([i1, i2], axis=1)
    e2 = jnp.exp(m2 - m1)
    g1 = 1.0 / (1.0 + e2)
    gates_ref[...] = jnp.concatenate([g1, e2 * g1], axis=1)


@jax.jit
def kernel(x, W, b):
    B, S, D = x.shape
    n = B * S
    x2 = x.reshape(n, D)
    b2 = b.reshape(1, NUM_EXPERTS)

    grid = (n // _BLOCK,)
    logits, idx, gates = pl.pallas_call(
        _router_block,
        grid=grid,
        in_specs=[
            pl.BlockSpec((_BLOCK, D), lambda i: (i, 0)),
            pl.BlockSpec((D, NUM_EXPERTS), lambda i: (0, 0)),
            pl.BlockSpec((1, NUM_EXPERTS), lambda i: (0, 0)),
        ],
        out_specs=[
            pl.BlockSpec((_BLOCK, NUM_EXPERTS), lambda i: (i, 0)),
            pl.BlockSpec((_BLOCK, TOP_K), lambda i: (i, 0)),
            pl.BlockSpec((_BLOCK, TOP_K), lambda i: (i, 0)),
        ],
        out_shape=[
            jax.ShapeDtypeStruct((n, NUM_EXPERTS), jnp.float32),
            jax.ShapeDtypeStruct((n, TOP_K), jnp.int32),
            jax.ShapeDtypeStruct((n, TOP_K), jnp.float32),
        ],
        compiler_params=pltpu.CompilerParams(
            dimension_semantics=("parallel",),
            vmem_limit_bytes=100 * 1024 * 1024,
        ),
    )(x2, W, b2)

    return (logits.reshape(B, S, NUM_EXPERTS),
            idx.reshape(B, S, TOP_K),
            gates.reshape(B, S, TOP_K))


# speedup vs baseline: 1.8692x; 1.0231x over previous
import functools

import jax
import jax.numpy as jnp
from jax.experimental import pallas as pl
from jax.experimental.pallas import tpu as pltpu

D_MODEL = 768
NUM_EXPERTS = 64
TOP_K = 2

_BLOCK = 4096


def _router_block(x_ref, w_ref, b_ref, logits_ref, idx_ref, gates_ref):
    x = x_ref[...]
    logits = jax.lax.dot_general(
        x, w_ref[...],
        dimension_numbers=(((1,), (1,)), ((), ())),
        preferred_element_type=jnp.float32)
    logits = logits + b_ref[...]
    logits_ref[...] = logits

    iota = jax.lax.broadcasted_iota(jnp.int32, logits.shape, 1)
    m1 = jnp.max(logits, axis=1, keepdims=True)
    i1 = jnp.min(jnp.where(logits == m1, iota, NUM_EXPERTS), axis=1,
                 keepdims=True)
    masked = jnp.where(iota == i1, -jnp.inf, logits)
    m2 = jnp.max(masked, axis=1, keepdims=True)
    i2 = jnp.min(jnp.where(masked == m2, iota, NUM_EXPERTS), axis=1,
                 keepdims=True)

    idx_ref[...] = jnp.concatenate([i1, i2], axis=1)
    e2 = jnp.exp(m2 - m1)
    g1 = 1.0 / (1.0 + e2)
    gates_ref[...] = jnp.concatenate([g1, e2 * g1], axis=1)


@jax.jit
def kernel(x, W, b):
    B, S, D = x.shape
    n = B * S
    x2 = x.reshape(n, D)
    Wt = W.T
    b2 = b.reshape(1, NUM_EXPERTS)

    grid = (n // _BLOCK,)
    logits, idx, gates = pl.pallas_call(
        _router_block,
        grid=grid,
        in_specs=[
            pl.BlockSpec((_BLOCK, D), lambda i: (i, 0)),
            pl.BlockSpec((NUM_EXPERTS, D), lambda i: (0, 0)),
            pl.BlockSpec((1, NUM_EXPERTS), lambda i: (0, 0)),
        ],
        out_specs=[
            pl.BlockSpec((_BLOCK, NUM_EXPERTS), lambda i: (i, 0)),
            pl.BlockSpec((_BLOCK, TOP_K), lambda i: (i, 0)),
            pl.BlockSpec((_BLOCK, TOP_K), lambda i: (i, 0)),
        ],
        out_shape=[
            jax.ShapeDtypeStruct((n, NUM_EXPERTS), jnp.float32),
            jax.ShapeDtypeStruct((n, TOP_K), jnp.int32),
            jax.ShapeDtypeStruct((n, TOP_K), jnp.float32),
        ],
        compiler_params=pltpu.CompilerParams(
            dimension_semantics=("parallel",),
            vmem_limit_bytes=100 * 1024 * 1024,
        ),
    )(x2, Wt, b2)

    return (logits.reshape(B, S, NUM_EXPERTS),
            idx.reshape(B, S, TOP_K),
            gates.reshape(B, S, TOP_K))
